# Initial kernel scaffold; baseline (speedup 1.0000x reference)
import functools
import math

import jax
import jax.numpy as jnp
from jax import lax
from jax.experimental import pallas as pl
from jax.experimental.pallas import tpu as pltpu
from jax.experimental.pallas import tpu_sc as plsc

N_NODES = 10000
N_EDGES = 320000
N_CLASSES = 10

NC, NS = 2, 16
NW = NC * NS
CHUNK = 128
NCH = 80
E_PAD = NW * NCH * CHUNK
N_ACC = 10240
ROWS_PER_TILE = N_ACC // NS

_mesh = lambda: plsc.VectorSubcoreMesh(core_axis_name="c", subcore_axis_name="s")


def _make_agg(d, nbuf=4):
  @functools.partial(
      pl.kernel,
      out_type=jax.ShapeDtypeStruct((NC, N_ACC, d), jnp.float32),
      mesh=_mesh(),
      scratch_types=[
          pltpu.VMEM((NCH, CHUNK), jnp.int32),
          pltpu.VMEM((NCH, CHUNK), jnp.int32),
          pltpu.VMEM((nbuf, CHUNK, d), jnp.float32),
          pltpu.VMEM_SHARED((N_ACC, d), jnp.float32),
          pltpu.SemaphoreType.DMA,
          pltpu.SemaphoreType.DMA,
      ],
  )
  def agg(hs, srcw, dstw, zinit, out, src_v, dst_v, bufs, acc, gsem, ssem):
    cid = lax.axis_index("c")
    sid = lax.axis_index("s")
    wid = cid * NS + sid
    pltpu.sync_copy(srcw.at[wid], src_v)
    pltpu.sync_copy(dstw.at[wid], dst_v)
    r0 = sid * ROWS_PER_TILE
    pltpu.sync_copy(zinit.at[pl.ds(r0, ROWS_PER_TILE)],
                    acc.at[pl.ds(r0, ROWS_PER_TILE)])
    plsc.subcore_barrier()

    def group(g, carry):
      base = g * nbuf
      gds = [pltpu.async_copy(hs.at[src_v.at[base + b]], bufs.at[b], gsem)
             for b in range(nbuf)]
      for gd in gds:
        gd.wait()
      sds = [pltpu.async_copy(bufs.at[b], acc.at[dst_v.at[base + b]], ssem,
                              add=True)
             for b in range(nbuf)]
      for sd in sds:
        sd.wait()
      return carry

    lax.fori_loop(0, NCH // nbuf, group, 0, unroll=False)
    plsc.subcore_barrier()
    pltpu.sync_copy(acc.at[pl.ds(r0, ROWS_PER_TILE)],
                    out.at[cid, pl.ds(r0, ROWS_PER_TILE)])

  return agg


def _make_deg(nbuf=8):
  @functools.partial(
      pl.kernel,
      out_type=jax.ShapeDtypeStruct((NC, N_ACC, 16), jnp.float32),
      mesh=_mesh(),
      scratch_types=[
          pltpu.VMEM((NCH, CHUNK), jnp.int32),
          pltpu.VMEM((CHUNK, 16), jnp.float32),
          pltpu.VMEM_SHARED((N_ACC, 16), jnp.float32),
          pltpu.SemaphoreType.DMA,
      ],
  )
  def deg(dstw, ones_c, zinit, out, dst_v, obuf, acc, ssem):
    cid = lax.axis_index("c")
    sid = lax.axis_index("s")
    wid = cid * NS + sid
    pltpu.sync_copy(dstw.at[wid], dst_v)
    pltpu.sync_copy(ones_c, obuf)
    r0 = sid * ROWS_PER_TILE
    pltpu.sync_copy(zinit.at[pl.ds(r0, ROWS_PER_TILE)],
                    acc.at[pl.ds(r0, ROWS_PER_TILE)])
    plsc.subcore_barrier()

    def group(g, carry):
      base = g * nbuf
      sds = [pltpu.async_copy(obuf, acc.at[dst_v.at[base + b]], ssem, add=True)
             for b in range(nbuf)]
      for sd in sds:
        sd.wait()
      return carry

    lax.fori_loop(0, NCH // nbuf, group, 0, unroll=False)
    plsc.subcore_barrier()
    pltpu.sync_copy(acc.at[pl.ds(r0, ROWS_PER_TILE)],
                    out.at[cid, pl.ds(r0, ROWS_PER_TILE)])

  return deg


_INV_SQRT2 = 1.0 / math.sqrt(2.0)


def _gelu(x):
  return 0.5 * x * (1.0 + lax.erf(x * _INV_SQRT2))


def _tc_first_body(degp_ref, x_ref, w_ref, dinv_ref, hs_ref):
  deg = degp_ref[0, :N_NODES] + degp_ref[1, :N_NODES] + 1.0
  dinv = lax.rsqrt(deg)
  dinv_ref[...] = dinv
  h = jnp.dot(x_ref[...], w_ref[...], preferred_element_type=jnp.float32)
  hs_ref[...] = h * dinv


def _tc_mid_body(d_in, d_out, p_ref, hs_ref, dinv_ref, b_ref, g_ref, be_ref,
                 w_ref, out_ref):
  dinv1 = dinv_ref[...][:, :1]
  agg = p_ref[0, :N_NODES] + p_ref[1, :N_NODES] + hs_ref[...]
  pre = agg * jnp.broadcast_to(dinv1, (N_NODES, d_in)) + b_ref[...]
  mean = jnp.mean(pre, axis=0, keepdims=True)
  var = jnp.mean((pre - mean) ** 2, axis=0, keepdims=True)
  xn = (pre - mean) * lax.rsqrt(var + 1e-5) * g_ref[...] + be_ref[...]
  act = _gelu(xn)
  h = jnp.dot(act, w_ref[...], preferred_element_type=jnp.float32)
  out_ref[...] = h * jnp.broadcast_to(dinv1, (N_NODES, d_out))


def _tc_final_body(p_ref, hs_ref, dinv_ref, b_ref, out_ref):
  agg = p_ref[0, :N_NODES] + p_ref[1, :N_NODES] + hs_ref[...]
  pre = agg * dinv_ref[...] + b_ref[...]
  logits = pre[:, :N_CLASSES]
  m = jnp.max(logits, axis=1, keepdims=True)
  lse = jnp.log(jnp.sum(jnp.exp(logits - m), axis=1, keepdims=True)) + m
  out_ref[...] = logits - lse


def _tc_first(degp, x, w):
  return pl.pallas_call(
      _tc_first_body,
      out_shape=[jax.ShapeDtypeStruct((N_NODES, 16), jnp.float32),
                 jax.ShapeDtypeStruct((N_NODES, 16), jnp.float32)],
  )(degp, x, w)


def _tc_mid(d_in, d_out, p, hs, dinv, b, g, be, w):
  return pl.pallas_call(
      functools.partial(_tc_mid_body, d_in, d_out),
      out_shape=jax.ShapeDtypeStruct((N_NODES, d_out), jnp.float32),
  )(p, hs, dinv, b, g, be, w)


def _tc_final(p, hs, dinv, b):
  return pl.pallas_call(
      _tc_final_body,
      out_shape=jax.ShapeDtypeStruct((N_NODES, N_CLASSES), jnp.float32),
  )(p, hs, dinv, b)


def kernel(x, edge_index, W1, b1, g1, be1, W2, b2, g2, be2, W3, b3, g3, be3,
           W4, b4):
  src = edge_index[0].astype(jnp.int32)
  dst = edge_index[1].astype(jnp.int32)
  pad = E_PAD - N_EDGES
  srcw = jnp.concatenate([src, jnp.zeros((pad,), jnp.int32)]
                         ).reshape(NW, NCH, CHUNK)
  dstw = jnp.concatenate([dst, jnp.full((pad,), N_NODES, jnp.int32)]
                         ).reshape(NW, NCH, CHUNK)

  ones_c = jnp.ones((CHUNK, 16), jnp.float32)
  z16 = jnp.zeros((N_ACC, 16), jnp.float32)
  z32 = jnp.zeros((N_ACC, 32), jnp.float32)
  z64 = jnp.zeros((N_ACC, 64), jnp.float32)

  w4p = jnp.pad(W4, ((0, 0), (0, 16 - N_CLASSES)))
  b4p = jnp.pad(b4, (0, 16 - N_CLASSES)).reshape(1, 16)
  row = lambda v: v.reshape(1, -1)

  degp = _make_deg()(dstw, ones_c, z16)
  dinv, hs1 = _tc_first(degp, x, W1)

  p1 = _make_agg(16)(hs1, srcw, dstw, z16)
  hs2 = _tc_mid(16, 32, p1, hs1, dinv, row(b1), row(g1), row(be1), W2)

  p2 = _make_agg(32)(hs2, srcw, dstw, z32)
  hs3 = _tc_mid(32, 64, p2, hs2, dinv, row(b2), row(g2), row(be2), W3)

  p3 = _make_agg(64)(hs3, srcw, dstw, z64)
  hs4 = _tc_mid(64, 16, p3, hs3, dinv, row(b3), row(g3), row(be3), w4p)

  p4 = _make_agg(16)(hs4, srcw, dstw, z16)
  return _tc_final(p4, hs4, dinv, b4p)


# speedup vs baseline: 18.6969x; 18.6969x over previous
import functools
import math

import jax
import jax.numpy as jnp
from jax import lax
from jax.experimental import pallas as pl
from jax.experimental.pallas import tpu as pltpu
from jax.experimental.pallas import tpu_sc as plsc

N_NODES = 10000
N_EDGES = 320000
N_CLASSES = 10

NC, NS = 2, 16
NW = NC * NS
CHUNK = 128
NCH = 80
E_PAD = NW * NCH * CHUNK
N_ACC = 10240
ROWS_PER_TILE = N_ACC // NS

_mesh = lambda: plsc.VectorSubcoreMesh(core_axis_name="c", subcore_axis_name="s")


def _make_agg(d, nbuf=4):
  @functools.partial(
      pl.kernel,
      out_type=jax.ShapeDtypeStruct((NC, N_ACC, d), jnp.float32),
      mesh=_mesh(),
      compiler_params=pltpu.CompilerParams(use_tc_tiling_on_sc=False),
      scratch_types=[
          pltpu.VMEM((NCH, CHUNK), jnp.int32),
          pltpu.VMEM((NCH, CHUNK), jnp.int32),
          pltpu.VMEM((nbuf, CHUNK, d), jnp.float32),
          pltpu.VMEM_SHARED((N_ACC, d), jnp.float32),
          pltpu.SemaphoreType.DMA,
          pltpu.SemaphoreType.DMA,
      ],
  )
  def agg(hs, srcw, dstw, zinit, out, src_v, dst_v, bufs, acc, gsem, ssem):
    cid = lax.axis_index("c")
    sid = lax.axis_index("s")
    wid = cid * NS + sid
    pltpu.sync_copy(srcw.at[wid], src_v)
    pltpu.sync_copy(dstw.at[wid], dst_v)
    r0 = sid * ROWS_PER_TILE
    pltpu.sync_copy(zinit.at[pl.ds(r0, ROWS_PER_TILE)],
                    acc.at[pl.ds(r0, ROWS_PER_TILE)])
    plsc.subcore_barrier()

    def group(g, carry):
      base = g * nbuf
      gds = [pltpu.async_copy(hs.at[src_v.at[base + b]], bufs.at[b], gsem)
             for b in range(nbuf)]
      for gd in gds:
        gd.wait()
      sds = [pltpu.async_copy(bufs.at[b], acc.at[dst_v.at[base + b]], ssem,
                              add=True)
             for b in range(nbuf)]
      for sd in sds:
        sd.wait()
      return carry

    lax.fori_loop(0, NCH // nbuf, group, 0, unroll=False)
    plsc.subcore_barrier()
    pltpu.sync_copy(acc.at[pl.ds(r0, ROWS_PER_TILE)],
                    out.at[cid, pl.ds(r0, ROWS_PER_TILE)])

  return agg


def _make_deg(nbuf=8):
  @functools.partial(
      pl.kernel,
      out_type=jax.ShapeDtypeStruct((NC, N_ACC, 16), jnp.float32),
      mesh=_mesh(),
      compiler_params=pltpu.CompilerParams(use_tc_tiling_on_sc=False),
      scratch_types=[
          pltpu.VMEM((NCH, CHUNK), jnp.int32),
          pltpu.VMEM((CHUNK, 16), jnp.float32),
          pltpu.VMEM_SHARED((N_ACC, 16), jnp.float32),
          pltpu.SemaphoreType.DMA,
      ],
  )
  def deg(dstw, ones_c, zinit, out, dst_v, obuf, acc, ssem):
    cid = lax.axis_index("c")
    sid = lax.axis_index("s")
    wid = cid * NS + sid
    pltpu.sync_copy(dstw.at[wid], dst_v)
    pltpu.sync_copy(ones_c, obuf)
    r0 = sid * ROWS_PER_TILE
    pltpu.sync_copy(zinit.at[pl.ds(r0, ROWS_PER_TILE)],
                    acc.at[pl.ds(r0, ROWS_PER_TILE)])
    plsc.subcore_barrier()

    def group(g, carry):
      base = g * nbuf
      sds = [pltpu.async_copy(obuf, acc.at[dst_v.at[base + b]], ssem, add=True)
             for b in range(nbuf)]
      for sd in sds:
        sd.wait()
      return carry

    lax.fori_loop(0, NCH // nbuf, group, 0, unroll=False)
    plsc.subcore_barrier()
    pltpu.sync_copy(acc.at[pl.ds(r0, ROWS_PER_TILE)],
                    out.at[cid, pl.ds(r0, ROWS_PER_TILE)])

  return deg


_INV_SQRT2 = 1.0 / math.sqrt(2.0)


def _gelu(x):
  return 0.5 * x * (1.0 + lax.erf(x * _INV_SQRT2))


def _tc_first_body(degp_ref, x_ref, w_ref, dinv_ref, hs_ref):
  deg = degp_ref[0, :N_NODES] + degp_ref[1, :N_NODES] + 1.0
  dinv = lax.rsqrt(deg)
  dinv_ref[...] = dinv
  h = jnp.dot(x_ref[...], w_ref[...], preferred_element_type=jnp.float32)
  hs_ref[...] = h * dinv


def _tc_mid_body(d_in, d_out, p_ref, hs_ref, dinv_ref, b_ref, g_ref, be_ref,
                 w_ref, out_ref):
  dinv1 = dinv_ref[...][:, :1]
  agg = p_ref[0, :N_NODES] + p_ref[1, :N_NODES] + hs_ref[...]
  pre = agg * jnp.broadcast_to(dinv1, (N_NODES, d_in)) + b_ref[...]
  mean = jnp.mean(pre, axis=0, keepdims=True)
  var = jnp.mean((pre - mean) ** 2, axis=0, keepdims=True)
  xn = (pre - mean) * lax.rsqrt(var + 1e-5) * g_ref[...] + be_ref[...]
  act = _gelu(xn)
  h = jnp.dot(act, w_ref[...], preferred_element_type=jnp.float32)
  out_ref[...] = h * jnp.broadcast_to(dinv1, (N_NODES, d_out))


def _tc_final_body(p_ref, hs_ref, dinv_ref, b_ref, out_ref):
  agg = p_ref[0, :N_NODES] + p_ref[1, :N_NODES] + hs_ref[...]
  pre = agg * dinv_ref[...] + b_ref[...]
  logits = pre[:, :N_CLASSES]
  m = jnp.max(logits, axis=1, keepdims=True)
  lse = jnp.log(jnp.sum(jnp.exp(logits - m), axis=1, keepdims=True)) + m
  out_ref[...] = logits - lse


def _tc_first(degp, x, w):
  return pl.pallas_call(
      _tc_first_body,
      out_shape=[jax.ShapeDtypeStruct((N_NODES, 16), jnp.float32),
                 jax.ShapeDtypeStruct((N_NODES, 16), jnp.float32)],
  )(degp, x, w)


def _tc_mid(d_in, d_out, p, hs, dinv, b, g, be, w):
  return pl.pallas_call(
      functools.partial(_tc_mid_body, d_in, d_out),
      out_shape=jax.ShapeDtypeStruct((N_NODES, d_out), jnp.float32),
  )(p, hs, dinv, b, g, be, w)


def _tc_final(p, hs, dinv, b):
  return pl.pallas_call(
      _tc_final_body,
      out_shape=jax.ShapeDtypeStruct((N_NODES, N_CLASSES), jnp.float32),
  )(p, hs, dinv, b)


def kernel(x, edge_index, W1, b1, g1, be1, W2, b2, g2, be2, W3, b3, g3, be3,
           W4, b4):
  src = edge_index[0].astype(jnp.int32)
  dst = edge_index[1].astype(jnp.int32)
  pad = E_PAD - N_EDGES
  srcw = jnp.concatenate([src, jnp.zeros((pad,), jnp.int32)]
                         ).reshape(NW, NCH, CHUNK)
  dstw = jnp.concatenate([dst, jnp.full((pad,), N_NODES, jnp.int32)]
                         ).reshape(NW, NCH, CHUNK)

  ones_c = jnp.ones((CHUNK, 16), jnp.float32)
  z16 = jnp.zeros((N_ACC, 16), jnp.float32)
  z32 = jnp.zeros((N_ACC, 32), jnp.float32)
  z64 = jnp.zeros((N_ACC, 64), jnp.float32)

  w4p = jnp.pad(W4, ((0, 0), (0, 16 - N_CLASSES)))
  b4p = jnp.pad(b4, (0, 16 - N_CLASSES)).reshape(1, 16)
  row = lambda v: v.reshape(1, -1)

  degp = _make_deg()(dstw, ones_c, z16)
  dinv, hs1 = _tc_first(degp, x, W1)

  p1 = _make_agg(16)(hs1, srcw, dstw, z16)
  hs2 = _tc_mid(16, 32, p1, hs1, dinv, row(b1), row(g1), row(be1), W2)

  p2 = _make_agg(32)(hs2, srcw, dstw, z32)
  hs3 = _tc_mid(32, 64, p2, hs2, dinv, row(b2), row(g2), row(be2), W3)

  p3 = _make_agg(64)(hs3, srcw, dstw, z64)
  hs4 = _tc_mid(64, 16, p3, hs3, dinv, row(b3), row(g3), row(be3), w4p)

  p4 = _make_agg(16)(hs4, srcw, dstw, z16)
  return _tc_final(p4, hs4, dinv, b4p)


# speedup vs baseline: 20.6377x; 1.1038x over previous
import functools
import math

import jax
import jax.numpy as jnp
from jax import lax
from jax.experimental import pallas as pl
from jax.experimental.pallas import tpu as pltpu
from jax.experimental.pallas import tpu_sc as plsc

N_NODES = 10000
N_EDGES = 320000
N_CLASSES = 10

NC, NS = 2, 16
NW = NC * NS
CHUNK = 128
NCH = 80
E_PAD = NW * NCH * CHUNK
N_ACC = 10240
ROWS_PER_TILE = N_ACC // NS

_mesh = lambda: plsc.VectorSubcoreMesh(core_axis_name="c", subcore_axis_name="s")


def _make_agg(d):
  nbuf, pre = 8, 4
  ngrp = NCH // nbuf

  @functools.partial(
      pl.kernel,
      out_type=jax.ShapeDtypeStruct((NC, N_ACC, d), jnp.float32),
      mesh=_mesh(),
      compiler_params=pltpu.CompilerParams(use_tc_tiling_on_sc=False),
      scratch_types=[
          pltpu.VMEM((NCH, CHUNK), jnp.int32),
          pltpu.VMEM((NCH, CHUNK), jnp.int32),
          pltpu.VMEM((nbuf, CHUNK, d), jnp.float32),
          pltpu.VMEM_SHARED((N_ACC, d), jnp.float32),
          pltpu.SemaphoreType.DMA((nbuf,)),
          pltpu.SemaphoreType.DMA((nbuf,)),
      ],
  )
  def agg(hs, srcw, dstw, zinit, out, src_v, dst_v, bufs, acc, gsem, ssem):
    cid = lax.axis_index("c")
    sid = lax.axis_index("s")
    wid = cid * NS + sid
    pltpu.sync_copy(srcw.at[wid], src_v)
    pltpu.sync_copy(dstw.at[wid], dst_v)
    r0 = sid * ROWS_PER_TILE
    pltpu.sync_copy(zinit.at[pl.ds(r0, ROWS_PER_TILE)],
                    acc.at[pl.ds(r0, ROWS_PER_TILE)])
    plsc.subcore_barrier()

    def gather(j, b):
      pltpu.async_copy(hs.at[src_v.at[j]], bufs.at[b], gsem.at[b])

    def gather_wait(j, b):
      pltpu.make_async_copy(hs.at[src_v.at[j]], bufs.at[b], gsem.at[b]).wait()

    def scatter(j, b):
      pltpu.async_copy(bufs.at[b], acc.at[dst_v.at[j]], ssem.at[b], add=True)

    def scatter_wait(j, b):
      pltpu.make_async_copy(bufs.at[b], acc.at[dst_v.at[j]],
                            ssem.at[b]).wait()

    for b in range(pre):
      gather(b, b)

    def group(g, carry):
      for i in range(nbuf):
        j = g * nbuf + i
        gather_wait(j, i)
        scatter(j, i)
        bn = (i + pre) % nbuf
        if i < pre:
          @pl.when(g >= 1)
          def _():
            scatter_wait(j + pre - nbuf, bn)
          gather(j + pre, bn)
        else:
          @pl.when(g < ngrp - 1)
          def _():
            scatter_wait(j + pre - nbuf, bn)
            gather(j + pre, bn)
      return carry

    lax.fori_loop(0, ngrp, group, 0, unroll=False)
    for b in range(nbuf):
      scatter_wait(NCH - nbuf + b, b)
    plsc.subcore_barrier()
    pltpu.sync_copy(acc.at[pl.ds(r0, ROWS_PER_TILE)],
                    out.at[cid, pl.ds(r0, ROWS_PER_TILE)])

  return agg


def _make_deg(nbuf=8):
  @functools.partial(
      pl.kernel,
      out_type=jax.ShapeDtypeStruct((NC, N_ACC, 16), jnp.float32),
      mesh=_mesh(),
      compiler_params=pltpu.CompilerParams(use_tc_tiling_on_sc=False),
      scratch_types=[
          pltpu.VMEM((NCH, CHUNK), jnp.int32),
          pltpu.VMEM((CHUNK, 16), jnp.float32),
          pltpu.VMEM_SHARED((N_ACC, 16), jnp.float32),
          pltpu.SemaphoreType.DMA,
      ],
  )
  def deg(dstw, ones_c, zinit, out, dst_v, obuf, acc, ssem):
    cid = lax.axis_index("c")
    sid = lax.axis_index("s")
    wid = cid * NS + sid
    pltpu.sync_copy(dstw.at[wid], dst_v)
    pltpu.sync_copy(ones_c, obuf)
    r0 = sid * ROWS_PER_TILE
    pltpu.sync_copy(zinit.at[pl.ds(r0, ROWS_PER_TILE)],
                    acc.at[pl.ds(r0, ROWS_PER_TILE)])
    plsc.subcore_barrier()

    def group(g, carry):
      base = g * nbuf
      sds = [pltpu.async_copy(obuf, acc.at[dst_v.at[base + b]], ssem, add=True)
             for b in range(nbuf)]
      for sd in sds:
        sd.wait()
      return carry

    lax.fori_loop(0, NCH // nbuf, group, 0, unroll=False)
    plsc.subcore_barrier()
    pltpu.sync_copy(acc.at[pl.ds(r0, ROWS_PER_TILE)],
                    out.at[cid, pl.ds(r0, ROWS_PER_TILE)])

  return deg


_INV_SQRT2 = 1.0 / math.sqrt(2.0)


def _gelu(x):
  return 0.5 * x * (1.0 + lax.erf(x * _INV_SQRT2))


def _tc_first_body(degp_ref, x_ref, w_ref, dinv_ref, hs_ref):
  deg = degp_ref[0, :N_NODES] + degp_ref[1, :N_NODES] + 1.0
  dinv = lax.rsqrt(deg)
  dinv_ref[...] = dinv
  h = jnp.dot(x_ref[...], w_ref[...], preferred_element_type=jnp.float32)
  hs_ref[...] = h * dinv


def _tc_mid_body(d_in, d_out, p_ref, hs_ref, dinv_ref, b_ref, g_ref, be_ref,
                 w_ref, out_ref):
  dinv1 = dinv_ref[...][:, :1]
  agg = p_ref[0, :N_NODES] + p_ref[1, :N_NODES] + hs_ref[...]
  pre = agg * jnp.broadcast_to(dinv1, (N_NODES, d_in)) + b_ref[...]
  mean = jnp.mean(pre, axis=0, keepdims=True)
  var = jnp.mean((pre - mean) ** 2, axis=0, keepdims=True)
  xn = (pre - mean) * lax.rsqrt(var + 1e-5) * g_ref[...] + be_ref[...]
  act = _gelu(xn)
  h = jnp.dot(act, w_ref[...], preferred_element_type=jnp.float32)
  out_ref[...] = h * jnp.broadcast_to(dinv1, (N_NODES, d_out))


def _tc_final_body(p_ref, hs_ref, dinv_ref, b_ref, out_ref):
  agg = p_ref[0, :N_NODES] + p_ref[1, :N_NODES] + hs_ref[...]
  pre = agg * dinv_ref[...] + b_ref[...]
  logits = pre[:, :N_CLASSES]
  m = jnp.max(logits, axis=1, keepdims=True)
  lse = jnp.log(jnp.sum(jnp.exp(logits - m), axis=1, keepdims=True)) + m
  out_ref[...] = logits - lse


def _tc_first(degp, x, w):
  return pl.pallas_call(
      _tc_first_body,
      out_shape=[jax.ShapeDtypeStruct((N_NODES, 16), jnp.float32),
                 jax.ShapeDtypeStruct((N_NODES, 16), jnp.float32)],
  )(degp, x, w)


def _tc_mid(d_in, d_out, p, hs, dinv, b, g, be, w):
  return pl.pallas_call(
      functools.partial(_tc_mid_body, d_in, d_out),
      out_shape=jax.ShapeDtypeStruct((N_NODES, d_out), jnp.float32),
  )(p, hs, dinv, b, g, be, w)


def _tc_final(p, hs, dinv, b):
  return pl.pallas_call(
      _tc_final_body,
      out_shape=jax.ShapeDtypeStruct((N_NODES, N_CLASSES), jnp.float32),
  )(p, hs, dinv, b)


def kernel(x, edge_index, W1, b1, g1, be1, W2, b2, g2, be2, W3, b3, g3, be3,
           W4, b4):
  src = edge_index[0].astype(jnp.int32)
  dst = edge_index[1].astype(jnp.int32)
  pad = E_PAD - N_EDGES
  srcw = jnp.concatenate([src, jnp.zeros((pad,), jnp.int32)]
                         ).reshape(NW, NCH, CHUNK)
  dstw = jnp.concatenate([dst, jnp.full((pad,), N_NODES, jnp.int32)]
                         ).reshape(NW, NCH, CHUNK)

  ones_c = jnp.ones((CHUNK, 16), jnp.float32)
  z16 = jnp.zeros((N_ACC, 16), jnp.float32)
  z32 = jnp.zeros((N_ACC, 32), jnp.float32)
  z64 = jnp.zeros((N_ACC, 64), jnp.float32)

  w4p = jnp.pad(W4, ((0, 0), (0, 16 - N_CLASSES)))
  b4p = jnp.pad(b4, (0, 16 - N_CLASSES)).reshape(1, 16)
  row = lambda v: v.reshape(1, -1)

  degp = _make_deg()(dstw, ones_c, z16)
  dinv, hs1 = _tc_first(degp, x, W1)

  p1 = _make_agg(16)(hs1, srcw, dstw, z16)
  hs2 = _tc_mid(16, 32, p1, hs1, dinv, row(b1), row(g1), row(be1), W2)

  p2 = _make_agg(32)(hs2, srcw, dstw, z32)
  hs3 = _tc_mid(32, 64, p2, hs2, dinv, row(b2), row(g2), row(be2), W3)

  p3 = _make_agg(64)(hs3, srcw, dstw, z64)
  hs4 = _tc_mid(64, 16, p3, hs3, dinv, row(b3), row(g3), row(be3), w4p)

  p4 = _make_agg(16)(hs4, srcw, dstw, z16)
  return _tc_final(p4, hs4, dinv, b4p)
